# baseline (device time: 43739 ns/iter reference)
import functools

import jax
import jax.numpy as jnp
from jax import lax
from jax.experimental import pallas as pl
from jax.experimental.pallas import tpu as pltpu

N_DEV = 4
N_LAYERS = 3


def kernel(x, Win0, Wout0, Win1, Wout1, Win2, Wout2):
    b, d_in = x.shape
    _, h_dim = Win0.shape

    def body(x_ref, win0_ref, wout0_ref, win1_ref, wout1_ref, win2_ref,
             wout2_ref, out_ref, send_buf, recv_buf, send_sems, recv_sems):
        my = lax.axis_index("i")
        p1 = my ^ 1
        p2 = 3 - my

        barrier_sem = pltpu.get_barrier_semaphore()
        for nbr in (p1, p2):
            pl.semaphore_signal(barrier_sem, inc=1, device_id=(nbr,),
                                device_id_type=pl.DeviceIdType.MESH)
        pl.semaphore_wait(barrier_sem, 2)

        wins = [win0_ref, win1_ref, win2_ref]
        wouts = [wout0_ref, wout1_ref, wout2_ref]

        xs = x_ref[...]
        for l in range(N_LAYERS):
            partial = jnp.dot(xs, wins[l][...],
                              preferred_element_type=jnp.float32)
            for s, partner in enumerate((p1, p2)):
                idx = 2 * l + s
                send_buf[...] = partial
                rdma = pltpu.make_async_remote_copy(
                    src_ref=send_buf,
                    dst_ref=recv_buf.at[idx],
                    send_sem=send_sems.at[idx],
                    recv_sem=recv_sems.at[idx],
                    device_id=(partner,),
                    device_id_type=pl.DeviceIdType.MESH,
                )
                rdma.start()
                rdma.wait()
                partial = partial + recv_buf[idx]
            h_act = jnp.maximum(partial, 0.0)
            xs = jnp.dot(h_act, wouts[l][...],
                         preferred_element_type=jnp.float32)
        out_ref[...] = xs

        @functools.partial(pl.run_scoped, sem=pltpu.SemaphoreType.REGULAR)
        def _(sem):
            for nbr in (p1, p2):
                pl.semaphore_signal(sem, inc=1, device_id=(nbr,),
                                    device_id_type=pl.DeviceIdType.MESH)
            pl.semaphore_wait(sem, 2)

    return pl.pallas_call(
        body,
        out_shape=jax.ShapeDtypeStruct((b, d_in), jnp.float32),
        in_specs=[pl.BlockSpec(memory_space=pltpu.VMEM)] * 7,
        out_specs=pl.BlockSpec(memory_space=pltpu.VMEM),
        scratch_shapes=[
            pltpu.VMEM((b, h_dim), jnp.float32),
            pltpu.VMEM((2 * N_LAYERS, b, h_dim), jnp.float32),
            pltpu.SemaphoreType.DMA((2 * N_LAYERS,)),
            pltpu.SemaphoreType.DMA((2 * N_LAYERS,)),
        ],
        compiler_params=pltpu.CompilerParams(collective_id=0),
    )(x, Win0, Wout0, Win1, Wout1, Win2, Wout2)


# device time: 39380 ns/iter; 1.1107x vs baseline; 1.1107x over previous
import functools

import jax
import jax.numpy as jnp
from jax import lax
from jax.experimental import pallas as pl
from jax.experimental.pallas import tpu as pltpu

N_DEV = 4
N_LAYERS = 3
N_SLOTS = 4 * N_LAYERS


def kernel(x, Win0, Wout0, Win1, Wout1, Win2, Wout2):
    b, d_in = x.shape
    _, h_dim = Win0.shape
    h2 = h_dim // 2

    def body(x_ref, win0_ref, wout0_ref, win1_ref, wout1_ref, win2_ref,
             wout2_ref, out_ref, send_buf, recv_buf, send_sems, recv_sems):
        my = lax.axis_index("i")
        p1 = my ^ 1
        p2 = 3 - my

        barrier_sem = pltpu.get_barrier_semaphore()
        for nbr in (p1, p2):
            pl.semaphore_signal(barrier_sem, inc=1, device_id=(nbr,),
                                device_id_type=pl.DeviceIdType.MESH)
        pl.semaphore_wait(barrier_sem, 2)

        def make(idx, partner):
            return pltpu.make_async_remote_copy(
                src_ref=send_buf.at[idx],
                dst_ref=recv_buf.at[idx],
                send_sem=send_sems.at[idx],
                recv_sem=recv_sems.at[idx],
                device_id=(partner,),
                device_id_type=pl.DeviceIdType.MESH,
            )

        wins = [win0_ref, win1_ref, win2_ref]
        wouts = [wout0_ref, wout1_ref, wout2_ref]

        xs = x_ref[...]
        for l in range(N_LAYERS):
            base = 4 * l
            iA1, iB1, iA2, iB2 = base, base + 1, base + 2, base + 3
            win, wout = wins[l], wouts[l]

            pA = jnp.dot(xs, win[:, :h2], preferred_element_type=jnp.float32)
            send_buf[iA1] = pA
            rA1 = make(iA1, p1)
            rA1.start()
            pB = jnp.dot(xs, win[:, h2:], preferred_element_type=jnp.float32)
            send_buf[iB1] = pB
            rB1 = make(iB1, p1)
            rB1.start()

            rA1.wait_recv()
            sA = pA + recv_buf[iA1]
            send_buf[iA2] = sA
            rA2 = make(iA2, p2)
            rA2.start()

            rB1.wait_recv()
            sB = pB + recv_buf[iB1]
            send_buf[iB2] = sB
            rB2 = make(iB2, p2)
            rB2.start()

            rA2.wait_recv()
            hA = jnp.maximum(sA + recv_buf[iA2], 0.0)
            xn = jnp.dot(hA, wout[:h2, :], preferred_element_type=jnp.float32)

            rB2.wait_recv()
            hB = jnp.maximum(sB + recv_buf[iB2], 0.0)
            xs = xn + jnp.dot(hB, wout[h2:, :],
                              preferred_element_type=jnp.float32)

            rA1.wait_send()
            rB1.wait_send()
            rA2.wait_send()
            rB2.wait_send()
        out_ref[...] = xs

        @functools.partial(pl.run_scoped, sem=pltpu.SemaphoreType.REGULAR)
        def _(sem):
            for nbr in (p1, p2):
                pl.semaphore_signal(sem, inc=1, device_id=(nbr,),
                                    device_id_type=pl.DeviceIdType.MESH)
            pl.semaphore_wait(sem, 2)

    return pl.pallas_call(
        body,
        out_shape=jax.ShapeDtypeStruct((b, d_in), jnp.float32),
        in_specs=[pl.BlockSpec(memory_space=pltpu.VMEM)] * 7,
        out_specs=pl.BlockSpec(memory_space=pltpu.VMEM),
        scratch_shapes=[
            pltpu.VMEM((N_SLOTS, b, h2), jnp.float32),
            pltpu.VMEM((N_SLOTS, b, h2), jnp.float32),
            pltpu.SemaphoreType.DMA((N_SLOTS,)),
            pltpu.SemaphoreType.DMA((N_SLOTS,)),
        ],
        compiler_params=pltpu.CompilerParams(collective_id=0),
    )(x, Win0, Wout0, Win1, Wout1, Win2, Wout2)


# device time: 32983 ns/iter; 1.3261x vs baseline; 1.1939x over previous
import functools

import jax
import jax.numpy as jnp
from jax import lax
from jax.experimental import pallas as pl
from jax.experimental.pallas import tpu as pltpu

N_DEV = 4
N_LAYERS = 3
N_SLOTS = 4 * N_LAYERS


def kernel(x, Win0, Wout0, Win1, Wout1, Win2, Wout2):
    b, d_in = x.shape
    _, h_dim = Win0.shape
    h2 = h_dim // 2

    def body(x_ref, win0_ref, wout0_ref, win1_ref, wout1_ref, win2_ref,
             wout2_ref, out_ref, send_buf, recv_buf, send_sems, recv_sems):
        my = lax.axis_index("i")
        p1 = my ^ 1
        p2 = 3 - my

        barrier_sem = pltpu.get_barrier_semaphore()
        for nbr in (p1, p2):
            pl.semaphore_signal(barrier_sem, inc=1, device_id=(nbr,),
                                device_id_type=pl.DeviceIdType.MESH)
        pl.semaphore_wait(barrier_sem, 2)

        def make(idx, partner):
            return pltpu.make_async_remote_copy(
                src_ref=send_buf.at[idx],
                dst_ref=recv_buf.at[idx],
                send_sem=send_sems.at[idx],
                recv_sem=recv_sems.at[idx],
                device_id=(partner,),
                device_id_type=pl.DeviceIdType.MESH,
            )

        wins = [win0_ref, win1_ref, win2_ref]
        wouts = [wout0_ref, wout1_ref, wout2_ref]

        xs = x_ref[...].astype(jnp.bfloat16)
        for l in range(N_LAYERS):
            base = 4 * l
            iA1, iB1, iA2, iB2 = base, base + 1, base + 2, base + 3
            win, wout = wins[l], wouts[l]

            pA = jnp.dot(xs, win[:, :h2].astype(jnp.bfloat16),
                         preferred_element_type=jnp.float32)
            send_buf[iA1] = pA.astype(jnp.bfloat16)
            rA1 = make(iA1, p1)
            rA1.start()
            pB = jnp.dot(xs, win[:, h2:].astype(jnp.bfloat16),
                         preferred_element_type=jnp.float32)
            send_buf[iB1] = pB.astype(jnp.bfloat16)
            rB1 = make(iB1, p1)
            rB1.start()

            rA1.wait_recv()
            sA = pA + recv_buf[iA1].astype(jnp.float32)
            send_buf[iA2] = sA.astype(jnp.bfloat16)
            rA2 = make(iA2, p2)
            rA2.start()

            rB1.wait_recv()
            sB = pB + recv_buf[iB1].astype(jnp.float32)
            send_buf[iB2] = sB.astype(jnp.bfloat16)
            rB2 = make(iB2, p2)
            rB2.start()

            rA2.wait_recv()
            hA = jnp.maximum(sA + recv_buf[iA2].astype(jnp.float32), 0.0)
            xn = jnp.dot(hA.astype(jnp.bfloat16),
                         wout[:h2, :].astype(jnp.bfloat16),
                         preferred_element_type=jnp.float32)

            rB2.wait_recv()
            hB = jnp.maximum(sB + recv_buf[iB2].astype(jnp.float32), 0.0)
            xs = (xn + jnp.dot(hB.astype(jnp.bfloat16),
                               wout[h2:, :].astype(jnp.bfloat16),
                               preferred_element_type=jnp.float32)
                  ).astype(jnp.bfloat16)

            rA1.wait_send()
            rB1.wait_send()
            rA2.wait_send()
            rB2.wait_send()
        out_ref[...] = xs.astype(jnp.float32)

        @functools.partial(pl.run_scoped, sem=pltpu.SemaphoreType.REGULAR)
        def _(sem):
            for nbr in (p1, p2):
                pl.semaphore_signal(sem, inc=1, device_id=(nbr,),
                                    device_id_type=pl.DeviceIdType.MESH)
            pl.semaphore_wait(sem, 2)

    return pl.pallas_call(
        body,
        out_shape=jax.ShapeDtypeStruct((b, d_in), jnp.float32),
        in_specs=[pl.BlockSpec(memory_space=pltpu.VMEM)] * 7,
        out_specs=pl.BlockSpec(memory_space=pltpu.VMEM),
        scratch_shapes=[
            pltpu.VMEM((N_SLOTS, b, h2), jnp.bfloat16),
            pltpu.VMEM((N_SLOTS, b, h2), jnp.bfloat16),
            pltpu.SemaphoreType.DMA((N_SLOTS,)),
            pltpu.SemaphoreType.DMA((N_SLOTS,)),
        ],
        compiler_params=pltpu.CompilerParams(collective_id=0),
    )(x, Win0, Wout0, Win1, Wout1, Win2, Wout2)
